# baseline (speedup 1.0000x reference)
import functools

import jax
import jax.numpy as jnp
import numpy as np
from jax.experimental import pallas as pl
from jax.experimental.pallas import tpu as pltpu

KS = 128
GF = 0.1

_ROWS = 512


def _step_table(length: int) -> np.ndarray:
    vals = []
    for ml in range(length + 1):
        max_tokens = max(1, int(round(GF * ml)))
        vals.append(max(1, int(round(ml / max_tokens))))
    return np.asarray(vals, dtype=np.int32)


def _mask_body(seq_ref, table_ref, out_ref, *, nb: int, length: int):
    max_len = seq_ref[0]
    for b in range(1, nb):
        max_len = jnp.maximum(max_len, seq_ref[b])
    step = table_ref[max_len]

    i0 = pl.program_id(0) * _ROWS
    rows = jax.lax.broadcasted_iota(jnp.int32, (_ROWS, length), 0) + i0
    cols = jax.lax.broadcasted_iota(jnp.int32, (_ROWS, length), 1)

    band = jnp.logical_and(cols <= rows, cols >= rows - KS)
    gi = jnp.logical_and(rows < max_len, rows % step == 0)
    gj = jnp.logical_and(cols < max_len, cols % step == 0)
    out_ref[...] = jnp.logical_not(jnp.logical_or(band, jnp.logical_or(gi, gj)))


def kernel(x, timestamps, seq_lens):
    length = x.shape[1]
    nb = seq_lens.shape[0]
    table = jnp.asarray(_step_table(length))

    mask = pl.pallas_call(
        functools.partial(_mask_body, nb=nb, length=length),
        grid=(length // _ROWS,),
        in_specs=[
            pl.BlockSpec(memory_space=pltpu.SMEM),
            pl.BlockSpec(memory_space=pltpu.SMEM),
        ],
        out_specs=pl.BlockSpec((_ROWS, length), lambda i: (i, 0)),
        out_shape=jax.ShapeDtypeStruct((length, length), jnp.bool_),
    )(seq_lens.astype(jnp.int32), table)

    return (x, timestamps, mask)
